# baseline (speedup 1.0000x reference)
import functools

import jax
import jax.numpy as jnp
from jax import lax
from jax.experimental import pallas as pl
from jax.experimental.pallas import tpu as pltpu
from jax.experimental.pallas import tpu_sc as plsc

_NC = 2
_NS = 16
_NW = _NC * _NS

_YT = 96


def _prep_body(C, oW, upd_ref, mask_ref, updt_ref, idx_ref):
    u = upd_ref[0]
    m = mask_ref[0]
    r = m // C
    updt_ref[0] = u.T
    idx_ref[0] = r.T


def _sc_scatter_body(C, P, oW, NPASS, TASKS, CH,
                     idx_hbm, upd_hbm, out_hbm,
                     idx_v, vb0, vb1, acc_a, acc_b,
                     sem_in, sem_v0, sem_v1, sem_wa, sem_wb):
    wid = lax.axis_index("s") * _NC + lax.axis_index("c")
    zero16 = jnp.zeros((16,), jnp.float32)

    ACC = _YT * oW
    NCH = P // CH
    accs = (acc_a, acc_b)
    wsems = (sem_wa, sem_wb)
    vbs = (vb0, vb1)
    vsems = (sem_v0, sem_v1)
    pending = [False, False]

    for t in range(TASKS):
        tid = wid * TASKS + t
        b = tid // C
        c = tid % C
        in_h = pltpu.async_copy(idx_hbm.at[b, c], idx_v, sem_in)
        for p in range(NPASS):
            base = p * ACC
            y0 = p * _YT
            a = p % 2
            acc = accs[a]
            wsem = wsems[a]

            pltpu.async_copy(upd_hbm.at[b, c, pl.ds(0, CH)], vb0, sem_v0)

            if pending[a]:
                def dbody(rr, z):
                    pltpu.make_async_copy(acc.at[pl.ds(0, oW)],
                                          out_hbm.at[b, y0, c], wsem).wait()
                    return z

                lax.fori_loop(0, _YT, dbody, 0)

            def zbody(i, z):
                acc[pl.ds(i * 16, 16)] = zero16
                return z

            lax.fori_loop(0, ACC // 16, zbody, 0, unroll=8)
            if p == 0:
                in_h.wait()

            def scatter_chunk(ck, vb):
                def sbody(i, z):
                    g0 = ck * CH + i * 64
                    i0 = i * 64
                    rs = [idx_v[pl.ds(g0 + u * 16, 16)] for u in range(4)]
                    vs = [vb[pl.ds(i0 + u * 16, 16)] for u in range(4)]
                    for u in range(4):
                        off = rs[u] - base
                        ok = (off >= 0) & (off < ACC)
                        plsc.addupdate_scatter(acc, [off], vs[u], mask=ok)
                    return z

                lax.fori_loop(0, CH // 64, sbody, 0)

            def cpair(k2, z):
                c0 = 2 * k2
                c1 = c0 + 1
                pltpu.async_copy(upd_hbm.at[b, c, pl.ds(c1 * CH, CH)],
                                 vb1, sem_v1)
                pltpu.make_async_copy(upd_hbm.at[b, c, pl.ds(0, CH)],
                                      vb0, sem_v0).wait()
                scatter_chunk(c0, vb0)
                c2 = jnp.minimum(c1 + 1, NCH - 1)
                pltpu.async_copy(upd_hbm.at[b, c, pl.ds(c2 * CH, CH)],
                                 vb0, sem_v0)
                pltpu.make_async_copy(upd_hbm.at[b, c, pl.ds(0, CH)],
                                      vb1, sem_v1).wait()
                scatter_chunk(c1, vb1)
                return z

            lax.fori_loop(0, NCH // 2, cpair, 0)
            pltpu.make_async_copy(upd_hbm.at[b, c, pl.ds(0, CH)],
                                  vb0, sem_v0).wait()

            def wrow(rr, z):
                pltpu.async_copy(acc.at[pl.ds(rr * oW, oW)],
                                 out_hbm.at[b, y0 + rr, c], wsem)
                return z

            lax.fori_loop(0, _YT, wrow, 0)
            pending[a] = True

    for a in (0, 1):
        if pending[a]:
            def fdrain(rr, z):
                pltpu.make_async_copy(accs[a].at[pl.ds(0, oW)],
                                      out_hbm.at[0, 0, 0], wsems[a]).wait()
                return z

            lax.fori_loop(0, _YT, fdrain, 0)


def kernel(updates, mask):
    B, H, W, C = updates.shape
    size = (2, 2)
    oH, oW = H * size[0], W * size[1]
    P = H * W

    mask = mask.astype(jnp.int32)
    upd2 = updates.reshape(B, P, C)
    mask2 = mask.reshape(B, P, C)

    PT = 2304
    assert P % PT == 0
    n_pt = P // PT
    updt, idx = pl.pallas_call(
        functools.partial(_prep_body, C, oW),
        grid=(B, n_pt),
        in_specs=[
            pl.BlockSpec((1, PT, C), lambda b, i: (b, i, 0)),
            pl.BlockSpec((1, PT, C), lambda b, i: (b, i, 0)),
        ],
        out_specs=[
            pl.BlockSpec((1, C, PT), lambda b, i: (b, 0, i)),
            pl.BlockSpec((1, C, PT), lambda b, i: (b, 0, i)),
        ],
        out_shape=[
            jax.ShapeDtypeStruct((B, C, P), jnp.float32),
            jax.ShapeDtypeStruct((B, C, P), jnp.int32),
        ],
    )(upd2, mask2)

    NPASS = oH // _YT
    CH = 4608
    assert oH % _YT == 0 and P % 16 == 0 and oW % 16 == 0
    assert P % CH == 0 and (P // CH) % 2 == 0 and CH % 64 == 0
    assert (B * C) % _NW == 0
    TASKS = (B * C) // _NW

    mesh = plsc.VectorSubcoreMesh(core_axis_name="c", subcore_axis_name="s")
    sc_call = pl.kernel(
        functools.partial(_sc_scatter_body, C, P, oW, NPASS, TASKS, CH),
        out_type=jax.ShapeDtypeStruct((B, oH, C, oW), jnp.float32),
        mesh=mesh,
        scratch_types=[
            pltpu.VMEM((P,), jnp.int32),
            pltpu.VMEM((CH,), jnp.float32),
            pltpu.VMEM((CH,), jnp.float32),
            pltpu.VMEM((_YT * oW,), jnp.float32),
            pltpu.VMEM((_YT * oW,), jnp.float32),
            pltpu.SemaphoreType.DMA,
            pltpu.SemaphoreType.DMA,
            pltpu.SemaphoreType.DMA,
            pltpu.SemaphoreType.DMA,
            pltpu.SemaphoreType.DMA,
        ],
        compiler_params=pltpu.CompilerParams(needs_layout_passes=False),
    )
    out_ycx = sc_call(idx, updt)

    return jnp.swapaxes(out_ycx, 2, 3)


# speedup vs baseline: 1.2109x; 1.2109x over previous
import functools

import jax
import jax.numpy as jnp
from jax import lax
from jax.experimental import pallas as pl
from jax.experimental.pallas import tpu as pltpu
from jax.experimental.pallas import tpu_sc as plsc

_NC = 2
_NS = 16
_NW = _NC * _NS

_YT = 128


def _prep_body(C, oW, upd_ref, mask_ref, updt_ref, idx_ref):
    u = upd_ref[0]
    m = mask_ref[0]
    r = m // C
    updt_ref[0] = u.T
    idx_ref[0] = r.T


def _sc_scatter_body(C, P, oW, NPASS, TASKS,
                     idx_hbm, upd_hbm, out_hbm,
                     idx_v, val_v, acc_v, sem_in, sem_wb):
    wid = lax.axis_index("s") * _NC + lax.axis_index("c")
    zero16 = jnp.zeros((16,), jnp.float32)

    ACC = _YT * oW

    def task_body(t, carry):
        tid = wid * TASKS + t
        b = tid // C
        c = tid % C
        in_i = pltpu.async_copy(idx_hbm.at[b, c], idx_v, sem_in)
        in_v = pltpu.async_copy(upd_hbm.at[b, c], val_v, sem_in)
        for p in range(NPASS):
            base = p * ACC
            y0 = p * _YT

            def zbody(i, z):
                acc_v[pl.ds(i * 16, 16)] = zero16
                return z

            lax.fori_loop(0, ACC // 16, zbody, 0, unroll=8)
            if p == 0:
                in_i.wait()
                in_v.wait()

            def sbody(i, z):
                i0 = i * 64
                rs = [idx_v[pl.ds(i0 + u * 16, 16)] for u in range(4)]
                vs = [val_v[pl.ds(i0 + u * 16, 16)] for u in range(4)]
                for u in range(4):
                    off = rs[u] - base
                    ok = (off >= 0) & (off < ACC)
                    plsc.addupdate_scatter(acc_v, [off], vs[u], mask=ok)
                return z

            lax.fori_loop(0, P // 64, sbody, 0, unroll=2)

            def wrow(rr, z):
                pltpu.async_copy(acc_v.at[pl.ds(rr * oW, oW)],
                                 out_hbm.at[b, y0 + rr, c], sem_wb)
                return z

            lax.fori_loop(0, _YT, wrow, 0)

            def wdrain(rr, z):
                pltpu.make_async_copy(acc_v.at[pl.ds(0, oW)],
                                      out_hbm.at[b, y0, c], sem_wb).wait()
                return z

            lax.fori_loop(0, _YT, wdrain, 0)
        return carry

    lax.fori_loop(0, TASKS, task_body, 0)


def kernel(updates, mask):
    B, H, W, C = updates.shape
    size = (2, 2)
    oH, oW = H * size[0], W * size[1]
    P = H * W

    mask = mask.astype(jnp.int32)
    upd2 = updates.reshape(B, P, C)
    mask2 = mask.reshape(B, P, C)

    PT = 2304
    assert P % PT == 0
    n_pt = P // PT
    updt, idx = pl.pallas_call(
        functools.partial(_prep_body, C, oW),
        grid=(B, n_pt),
        in_specs=[
            pl.BlockSpec((1, PT, C), lambda b, i: (b, i, 0)),
            pl.BlockSpec((1, PT, C), lambda b, i: (b, i, 0)),
        ],
        out_specs=[
            pl.BlockSpec((1, C, PT), lambda b, i: (b, 0, i)),
            pl.BlockSpec((1, C, PT), lambda b, i: (b, 0, i)),
        ],
        out_shape=[
            jax.ShapeDtypeStruct((B, C, P), jnp.float32),
            jax.ShapeDtypeStruct((B, C, P), jnp.int32),
        ],
    )(upd2, mask2)

    NPASS = oH // _YT
    assert oH % _YT == 0 and P % 64 == 0 and oW % 16 == 0
    assert (B * C) % _NW == 0
    TASKS = (B * C) // _NW

    mesh = plsc.VectorSubcoreMesh(core_axis_name="c", subcore_axis_name="s")
    sc_call = pl.kernel(
        functools.partial(_sc_scatter_body, C, P, oW, NPASS, TASKS),
        out_type=jax.ShapeDtypeStruct((B, oH, C, oW), jnp.float32),
        mesh=mesh,
        scratch_types=[
            pltpu.VMEM((P,), jnp.int32),
            pltpu.VMEM((P,), jnp.float32),
            pltpu.VMEM((_YT * oW,), jnp.float32),
            pltpu.SemaphoreType.DMA,
            pltpu.SemaphoreType.DMA,
        ],
        compiler_params=pltpu.CompilerParams(needs_layout_passes=False),
    )
    out_ycx = sc_call(idx, updt)

    return jnp.swapaxes(out_ycx, 2, 3)
